# Initial kernel scaffold; baseline (speedup 1.0000x reference)
import jax
import jax.numpy as jnp
from jax.experimental import pallas as pl


def kernel(dis_emb, dr_emb, latent_emb, di_lantent_weight, dr_lantent_weight, interact_mat, interact_mat_t, u_edge, v_edge, di_emb_sim, dr_emb_sim):
    raise NotImplementedError("write your pallas kernel here")


# speedup vs baseline: 1.1185x; 1.1185x over previous
import functools

import jax
import jax.numpy as jnp
from jax.experimental import pallas as pl

N_DIS = 4096
N_DRUG = 2048
DIM = 64
NFAC = 4
HOPS = 3
NSTEP1 = 16
NSTEP23 = 8


def _l2n(x):
    ss = jnp.sum(x * x, axis=1, keepdims=True)
    return x * jax.lax.rsqrt(jnp.maximum(ss, 1e-24))


def _hop_body(*refs, first, nstep):
    n_in = 13 if first else 10
    (a_ref, v_ref, u_ref, dis_ref, dr_ref, dsim_ref, usim_ref,
     dilw_ref, drlw_ref, lat_ref) = refs[:10]
    (dis_o, dr_o, dsim_o, usim_o,
     dis_no, dr_no, dsim_no, usim_no) = refs[n_in:n_in + 8]
    i = pl.program_id(0)
    bf16 = jnp.bfloat16
    f32 = jnp.float32
    hi = jax.lax.Precision.HIGHEST
    lat = lat_ref[...]
    a = a_ref[...].astype(bf16)
    dis_blk = dis_ref[...]
    dr_full = dr_ref[...]

    agg = jnp.dot(a, dr_full.astype(bf16), preferred_element_type=f32)
    scale = jnp.dot(dilw_ref[...], lat, precision=hi, preferred_element_type=f32) + 1.0
    dis_new = agg * scale
    dis_o[...] = dis_new
    dis_no[...] = _l2n(dis_new)

    contrib = jax.lax.dot_general(
        a, dis_blk.astype(bf16),
        (((0,), (0,)), ((), ())), preferred_element_type=f32)

    @pl.when(i == 0)
    def _():
        dr_o[...] = jnp.zeros_like(dr_o)

    dr_o[...] += contrib

    v = v_ref[...].astype(bf16)
    dsim_new = jnp.dot(v, dsim_ref[...].astype(bf16), preferred_element_type=f32)
    dsim_o[...] = dsim_new
    dsim_no[...] = _l2n(dsim_new)

    u = u_ref[...].astype(bf16)
    usim_new = jnp.dot(u, usim_ref[...].astype(bf16), preferred_element_type=f32)
    usim_o[...] = usim_new
    usim_no[...] = _l2n(usim_new)

    @pl.when(i == nstep - 1)
    def _():
        dscale = jnp.dot(drlw_ref[...], lat, precision=hi,
                         preferred_element_type=f32) + 1.0
        drn = dr_o[...] * dscale
        dr_o[...] = drn
        dr_no[...] = _l2n(drn)

    if first:
        dr_blk_ref, dsim_blk_ref, usim_blk_ref = refs[10:13]
        dis0_no, dr0_no, dsim0_no, usim0_no, a_bo, v_bo, u_bo = refs[n_in + 8:]
        dis0_no[...] = _l2n(dis_blk)
        dr0_no[...] = _l2n(dr_blk_ref[...])
        dsim0_no[...] = _l2n(dsim_blk_ref[...])
        usim0_no[...] = _l2n(usim_blk_ref[...])
        a_bo[...] = a
        v_bo[...] = v
        u_bo[...] = u


def _run_hop(a, v, u, dis, dr, dsim, usim, dilw, drlw, lat, first):
    nstep = NSTEP1 if first else NSTEP23
    DB = N_DIS // nstep
    UB = N_DRUG // nstep
    f32 = jnp.float32
    bf16 = jnp.bfloat16
    dis_blk_spec = pl.BlockSpec((DB, DIM), lambda i: (i, 0))
    drug_blk_spec = pl.BlockSpec((UB, DIM), lambda i: (i, 0))
    dr_res_spec = pl.BlockSpec((N_DRUG, DIM), lambda i: (0, 0))

    in_specs = [
        pl.BlockSpec((DB, N_DRUG), lambda i: (i, 0)),
        pl.BlockSpec((DB, N_DIS), lambda i: (i, 0)),
        pl.BlockSpec((UB, N_DRUG), lambda i: (i, 0)),
        dis_blk_spec,
        dr_res_spec,
        pl.BlockSpec((N_DIS, DIM), lambda i: (0, 0)),
        dr_res_spec,
        pl.BlockSpec((DB, NFAC), lambda i: (i, 0)),
        pl.BlockSpec((N_DRUG, NFAC), lambda i: (0, 0)),
        pl.BlockSpec((NFAC, DIM), lambda i: (0, 0)),
    ]
    operands = [a, v, u, dis, dr, dsim, usim, dilw, drlw, lat]
    if first:
        in_specs += [drug_blk_spec, dis_blk_spec, drug_blk_spec]
        operands += [dr, dsim, usim]
    out_shape = [
        jax.ShapeDtypeStruct((N_DIS, DIM), f32),
        jax.ShapeDtypeStruct((N_DRUG, DIM), f32),
        jax.ShapeDtypeStruct((N_DIS, DIM), f32),
        jax.ShapeDtypeStruct((N_DRUG, DIM), f32),
        jax.ShapeDtypeStruct((N_DIS, DIM), f32),
        jax.ShapeDtypeStruct((N_DRUG, DIM), f32),
        jax.ShapeDtypeStruct((N_DIS, DIM), f32),
        jax.ShapeDtypeStruct((N_DRUG, DIM), f32),
    ]
    out_specs = [
        dis_blk_spec, dr_res_spec, dis_blk_spec, drug_blk_spec,
        dis_blk_spec, dr_res_spec, dis_blk_spec, drug_blk_spec,
    ]
    if first:
        out_shape += [
            jax.ShapeDtypeStruct((N_DIS, DIM), f32),
            jax.ShapeDtypeStruct((N_DRUG, DIM), f32),
            jax.ShapeDtypeStruct((N_DIS, DIM), f32),
            jax.ShapeDtypeStruct((N_DRUG, DIM), f32),
            jax.ShapeDtypeStruct((N_DIS, N_DRUG), bf16),
            jax.ShapeDtypeStruct((N_DIS, N_DIS), bf16),
            jax.ShapeDtypeStruct((N_DRUG, N_DRUG), bf16),
        ]
        out_specs += [
            dis_blk_spec, drug_blk_spec, dis_blk_spec, drug_blk_spec,
            pl.BlockSpec((DB, N_DRUG), lambda i: (i, 0)),
            pl.BlockSpec((DB, N_DIS), lambda i: (i, 0)),
            pl.BlockSpec((UB, N_DRUG), lambda i: (i, 0)),
        ]
    return pl.pallas_call(
        functools.partial(_hop_body, first=first, nstep=nstep),
        grid=(nstep,),
        in_specs=in_specs,
        out_specs=out_specs,
        out_shape=out_shape,
    )(*operands)


def kernel(dis_emb, dr_emb, latent_emb, di_lantent_weight, dr_lantent_weight,
           interact_mat, interact_mat_t, u_edge, v_edge, di_emb_sim, dr_emb_sim):
    del interact_mat_t
    a, v, u = interact_mat, v_edge, u_edge
    dis, dr, dsim, usim = dis_emb, dr_emb, di_emb_sim, dr_emb_sim
    dis_parts, drug_parts = [], []
    for h in range(HOPS):
        outs = _run_hop(a, v, u, dis, dr, dsim, usim,
                        di_lantent_weight, dr_lantent_weight, latent_emb,
                        first=(h == 0))
        dis_new, dr_new, dsim_new, usim_new, dis_n, dr_n, dsim_n, usim_n = outs[:8]
        if h == 0:
            dis0_n, dr0_n, dsim0_n, usim0_n, a, v, u = outs[8:]
            dis_parts += [dis0_n, dsim0_n]
            drug_parts += [dr0_n, usim0_n]
        dis_parts += [dis_n, dsim_n]
        drug_parts += [dr_n, usim_n]
        dis, dr, dsim, usim = dis_new, dr_new, dsim_new, usim_new
    return (jnp.concatenate(dis_parts, axis=-1),
            jnp.concatenate(drug_parts, axis=-1),
            jnp.float32(0.0))


# speedup vs baseline: 1.1668x; 1.0432x over previous
import functools

import jax
import jax.numpy as jnp
from jax.experimental import pallas as pl

N_DIS = 4096
N_DRUG = 2048
DIM = 64
NFAC = 4
HOPS = 3
NSTEP1 = 16
NSTEP23 = 8
NSTEP_AS = 8


def _l2n(x):
    ss = jnp.sum(x * x, axis=1, keepdims=True)
    return x * jax.lax.rsqrt(jnp.maximum(ss, 1e-24))


def _hop_body(*refs, first, nstep):
    (a_ref, v_ref, u_ref, dis_ref, dr_ref, dsim_ref, usim_ref,
     dilw_ref, drlw_ref, lat_ref) = refs[:10]
    (dis_o, dr_o, dsim_o, usim_o,
     dis_bo, dr_bo, dsim_bo, usim_bo) = refs[10:18]
    i = pl.program_id(0)
    bf16 = jnp.bfloat16
    f32 = jnp.float32
    hi = jax.lax.Precision.HIGHEST
    lat = lat_ref[...]
    if first:
        a = a_ref[...].astype(bf16)
        dis_blk = dis_ref[...].astype(bf16)
        dr_full = dr_ref[...].astype(bf16)
        dsim_full = dsim_ref[...].astype(bf16)
        usim_full = usim_ref[...].astype(bf16)
    else:
        a = a_ref[...]
        dis_blk = dis_ref[...]
        dr_full = dr_ref[...]
        dsim_full = dsim_ref[...]
        usim_full = usim_ref[...]

    agg = jnp.dot(a, dr_full, preferred_element_type=f32)
    scale = jnp.dot(dilw_ref[...], lat, precision=hi, preferred_element_type=f32) + 1.0
    dis_new = agg * scale
    dis_o[...] = dis_new
    dis_bo[...] = dis_new.astype(bf16)

    contrib = jax.lax.dot_general(
        a, dis_blk, (((0,), (0,)), ((), ())), preferred_element_type=f32)

    @pl.when(i == 0)
    def _():
        dr_o[...] = jnp.zeros_like(dr_o)

    dr_o[...] += contrib

    dsim_new = jnp.dot(v_ref[...] if not first else v_ref[...].astype(bf16),
                       dsim_full, preferred_element_type=f32)
    dsim_o[...] = dsim_new
    dsim_bo[...] = dsim_new.astype(bf16)
    usim_new = jnp.dot(u_ref[...] if not first else u_ref[...].astype(bf16),
                       usim_full, preferred_element_type=f32)
    usim_o[...] = usim_new
    usim_bo[...] = usim_new.astype(bf16)

    @pl.when(i == nstep - 1)
    def _():
        dscale = jnp.dot(drlw_ref[...], lat, precision=hi,
                         preferred_element_type=f32) + 1.0
        drn = dr_o[...] * dscale
        dr_o[...] = drn
        dr_bo[...] = drn.astype(bf16)

    if first:
        a_bo, v_bo, u_bo = refs[18:21]
        a_bo[...] = a
        v_bo[...] = v_ref[...].astype(bf16)
        u_bo[...] = u_ref[...].astype(bf16)


def _run_hop(a, v, u, dis, dr, dsim, usim, dilw, drlw, lat, first):
    nstep = NSTEP1 if first else NSTEP23
    db = N_DIS // nstep
    ub = N_DRUG // nstep
    f32 = jnp.float32
    bf16 = jnp.bfloat16
    dis_blk_spec = pl.BlockSpec((db, DIM), lambda i: (i, 0))
    drug_blk_spec = pl.BlockSpec((ub, DIM), lambda i: (i, 0))
    dr_res_spec = pl.BlockSpec((N_DRUG, DIM), lambda i: (0, 0))

    in_specs = [
        pl.BlockSpec((db, N_DRUG), lambda i: (i, 0)),
        pl.BlockSpec((db, N_DIS), lambda i: (i, 0)),
        pl.BlockSpec((ub, N_DRUG), lambda i: (i, 0)),
        dis_blk_spec,
        dr_res_spec,
        pl.BlockSpec((N_DIS, DIM), lambda i: (0, 0)),
        dr_res_spec,
        pl.BlockSpec((db, NFAC), lambda i: (i, 0)),
        pl.BlockSpec((N_DRUG, NFAC), lambda i: (0, 0)),
        pl.BlockSpec((NFAC, DIM), lambda i: (0, 0)),
    ]
    out_shape = [
        jax.ShapeDtypeStruct((N_DIS, DIM), f32),
        jax.ShapeDtypeStruct((N_DRUG, DIM), f32),
        jax.ShapeDtypeStruct((N_DIS, DIM), f32),
        jax.ShapeDtypeStruct((N_DRUG, DIM), f32),
        jax.ShapeDtypeStruct((N_DIS, DIM), bf16),
        jax.ShapeDtypeStruct((N_DRUG, DIM), bf16),
        jax.ShapeDtypeStruct((N_DIS, DIM), bf16),
        jax.ShapeDtypeStruct((N_DRUG, DIM), bf16),
    ]
    out_specs = [dis_blk_spec, dr_res_spec, dis_blk_spec, drug_blk_spec,
                 dis_blk_spec, dr_res_spec, dis_blk_spec, drug_blk_spec]
    if first:
        out_shape += [
            jax.ShapeDtypeStruct((N_DIS, N_DRUG), bf16),
            jax.ShapeDtypeStruct((N_DIS, N_DIS), bf16),
            jax.ShapeDtypeStruct((N_DRUG, N_DRUG), bf16),
        ]
        out_specs += [
            pl.BlockSpec((db, N_DRUG), lambda i: (i, 0)),
            pl.BlockSpec((db, N_DIS), lambda i: (i, 0)),
            pl.BlockSpec((ub, N_DRUG), lambda i: (i, 0)),
        ]
    return pl.pallas_call(
        functools.partial(_hop_body, first=first, nstep=nstep),
        grid=(nstep,),
        in_specs=in_specs,
        out_specs=out_specs,
        out_shape=out_shape,
    )(a, v, u, dis, dr, dsim, usim, dilw, drlw, lat)


def _assemble_body(*refs):
    dis_refs = refs[:8]
    drug_refs = refs[8:16]
    dis_res_o, drug_res_o = refs[16:18]
    dis_res_o[...] = jnp.concatenate([_l2n(r[...]) for r in dis_refs], axis=1)
    drug_res_o[...] = jnp.concatenate([_l2n(r[...]) for r in drug_refs], axis=1)


def _assemble(dis_parts, drug_parts):
    db = N_DIS // NSTEP_AS
    ub = N_DRUG // NSTEP_AS
    f32 = jnp.float32
    dis_blk_spec = pl.BlockSpec((db, DIM), lambda i: (i, 0))
    drug_blk_spec = pl.BlockSpec((ub, DIM), lambda i: (i, 0))
    return pl.pallas_call(
        _assemble_body,
        grid=(NSTEP_AS,),
        in_specs=[dis_blk_spec] * 8 + [drug_blk_spec] * 8,
        out_specs=[pl.BlockSpec((db, 8 * DIM), lambda i: (i, 0)),
                   pl.BlockSpec((ub, 8 * DIM), lambda i: (i, 0))],
        out_shape=[jax.ShapeDtypeStruct((N_DIS, 8 * DIM), f32),
                   jax.ShapeDtypeStruct((N_DRUG, 8 * DIM), f32)],
    )(*dis_parts, *drug_parts)


def kernel(dis_emb, dr_emb, latent_emb, di_lantent_weight, dr_lantent_weight,
           interact_mat, interact_mat_t, u_edge, v_edge, di_emb_sim, dr_emb_sim):
    del interact_mat_t
    a, v, u = interact_mat, v_edge, u_edge
    dis, dr, dsim, usim = dis_emb, dr_emb, di_emb_sim, dr_emb_sim
    dis_parts, drug_parts = [dis, dsim], [dr, usim]
    for h in range(HOPS):
        outs = _run_hop(a, v, u, dis, dr, dsim, usim,
                        di_lantent_weight, dr_lantent_weight, latent_emb,
                        first=(h == 0))
        dis_f, dr_f, dsim_f, usim_f = outs[:4]
        dis, dr, dsim, usim = outs[4:8]
        if h == 0:
            a, v, u = outs[8:11]
        dis_parts += [dis_f, dsim_f]
        drug_parts += [dr_f, usim_f]
    dis_res, drug_res = _assemble(dis_parts, drug_parts)
    return (dis_res, drug_res, jnp.float32(0.0))


# speedup vs baseline: 1.4297x; 1.2253x over previous
import jax
import jax.numpy as jnp
from jax.experimental import pallas as pl

N_DIS = 4096
N_DRUG = 2048
DIM = 64
NFAC = 4
NSTEP1 = 16
NSTEP2 = 8
NSTEP3 = 8

_F32 = jnp.float32
_BF16 = jnp.bfloat16
_HI = jax.lax.Precision.HIGHEST


def _l2n(x):
    ss = jnp.sum(x * x, axis=1, keepdims=True)
    return x * jax.lax.rsqrt(jnp.maximum(ss, 1e-24))


def _dot_t(a, b):
    return jax.lax.dot_general(a, b, (((0,), (0,)), ((), ())),
                               preferred_element_type=_F32)


def _scale_of(w_ref, lat):
    return jnp.dot(w_ref[...], lat, precision=_HI,
                   preferred_element_type=_F32) + 1.0


def _hop1_body(a_ref, v_ref, u_ref, dis_ref, dr_ref, dsim_ref, usim_ref,
               dilw_ref, drlw_ref, lat_ref,
               dis_o, dr_o, dsim_o, usim_o,
               dis_bo, dr_bo, dsim_bo, usim_bo,
               a_bo, v_bo, u_bo):
    i = pl.program_id(0)
    lat = lat_ref[...]
    a = a_ref[...].astype(_BF16)
    v = v_ref[...].astype(_BF16)
    u = u_ref[...].astype(_BF16)

    dis_new = jnp.dot(a, dr_ref[...].astype(_BF16),
                      preferred_element_type=_F32) * _scale_of(dilw_ref, lat)
    dis_o[...] = dis_new
    dis_bo[...] = dis_new.astype(_BF16)

    @pl.when(i == 0)
    def _():
        dr_o[...] = jnp.zeros_like(dr_o)

    dr_o[...] += _dot_t(a, dis_ref[...].astype(_BF16))

    dsim_new = jnp.dot(v, dsim_ref[...].astype(_BF16), preferred_element_type=_F32)
    dsim_o[...] = dsim_new
    dsim_bo[...] = dsim_new.astype(_BF16)
    usim_new = jnp.dot(u, usim_ref[...].astype(_BF16), preferred_element_type=_F32)
    usim_o[...] = usim_new
    usim_bo[...] = usim_new.astype(_BF16)

    @pl.when(i == NSTEP1 - 1)
    def _():
        drn = dr_o[...] * _scale_of(drlw_ref, lat)
        dr_o[...] = drn
        dr_bo[...] = drn.astype(_BF16)

    a_bo[...] = a
    v_bo[...] = v
    u_bo[...] = u


def _hop2_body(a_ref, v_ref, u_ref, dis_ref, dr_ref, dsim_ref, usim_ref,
               dilw_ref, drlw_ref, lat_ref,
               dis_o, dr_o, dsim_o, usim_o, dr3_o,
               dr_bo, dsim_bo, usim_bo):
    i = pl.program_id(0)
    lat = lat_ref[...]
    a = a_ref[...]

    dis_new = jnp.dot(a, dr_ref[...],
                      preferred_element_type=_F32) * _scale_of(dilw_ref, lat)
    dis_o[...] = dis_new

    @pl.when(i == 0)
    def _():
        dr_o[...] = jnp.zeros_like(dr_o)
        dr3_o[...] = jnp.zeros_like(dr3_o)

    dr_o[...] += _dot_t(a, dis_ref[...])
    dr3_o[...] += _dot_t(a, dis_new.astype(_BF16))

    dsim_new = jnp.dot(v_ref[...], dsim_ref[...], preferred_element_type=_F32)
    dsim_o[...] = dsim_new
    dsim_bo[...] = dsim_new.astype(_BF16)
    usim_new = jnp.dot(u_ref[...], usim_ref[...], preferred_element_type=_F32)
    usim_o[...] = usim_new
    usim_bo[...] = usim_new.astype(_BF16)

    @pl.when(i == NSTEP2 - 1)
    def _():
        dscale = _scale_of(drlw_ref, lat)
        drn = dr_o[...] * dscale
        dr_o[...] = drn
        dr_bo[...] = drn.astype(_BF16)
        dr3_o[...] *= dscale


def _hop3_body(a_ref, v_ref, u_ref, dr2b_ref, dsim2b_ref, usim2b_ref,
               dilw_ref, lat_ref,
               dis0_ref, dsim0_ref, dis1_ref, dsim1_ref, dis2_ref, dsim2_ref,
               dr0_ref, usim0_ref, dr1_ref, usim1_ref, dr2_ref, usim2_ref,
               dr3_ref,
               dis_res_o, drug_res_o):
    lat = lat_ref[...]
    dis3 = jnp.dot(a_ref[...], dr2b_ref[...],
                   preferred_element_type=_F32) * _scale_of(dilw_ref, lat)
    dsim3 = jnp.dot(v_ref[...], dsim2b_ref[...], preferred_element_type=_F32)
    usim3 = jnp.dot(u_ref[...], usim2b_ref[...], preferred_element_type=_F32)

    dis_res_o[...] = jnp.concatenate(
        [_l2n(dis0_ref[...]), _l2n(dsim0_ref[...]),
         _l2n(dis1_ref[...]), _l2n(dsim1_ref[...]),
         _l2n(dis2_ref[...]), _l2n(dsim2_ref[...]),
         _l2n(dis3), _l2n(dsim3)], axis=1)
    drug_res_o[...] = jnp.concatenate(
        [_l2n(dr0_ref[...]), _l2n(usim0_ref[...]),
         _l2n(dr1_ref[...]), _l2n(usim1_ref[...]),
         _l2n(dr2_ref[...]), _l2n(usim2_ref[...]),
         _l2n(dr3_ref[...]), _l2n(usim3)], axis=1)


def kernel(dis_emb, dr_emb, latent_emb, di_lantent_weight, dr_lantent_weight,
           interact_mat, interact_mat_t, u_edge, v_edge, di_emb_sim, dr_emb_sim):
    del interact_mat_t
    dilw, drlw, lat = di_lantent_weight, dr_lantent_weight, latent_emb

    def dis_blk(n):
        return pl.BlockSpec((N_DIS // n, DIM), lambda i: (i, 0))

    def drug_blk(n):
        return pl.BlockSpec((N_DRUG // n, DIM), lambda i: (i, 0))

    def res(rows):
        return pl.BlockSpec((rows, DIM), lambda i: (0, 0))

    def shp(r, c, dt=_F32):
        return jax.ShapeDtypeStruct((r, c), dt)

    w_specs = [
        pl.BlockSpec((N_DIS // NSTEP1, NFAC), lambda i: (i, 0)),
        pl.BlockSpec((N_DRUG, NFAC), lambda i: (0, 0)),
        pl.BlockSpec((NFAC, DIM), lambda i: (0, 0)),
    ]

    db1, ub1 = N_DIS // NSTEP1, N_DRUG // NSTEP1
    outs1 = pl.pallas_call(
        _hop1_body,
        grid=(NSTEP1,),
        in_specs=[
            pl.BlockSpec((db1, N_DRUG), lambda i: (i, 0)),
            pl.BlockSpec((db1, N_DIS), lambda i: (i, 0)),
            pl.BlockSpec((ub1, N_DRUG), lambda i: (i, 0)),
            dis_blk(NSTEP1), res(N_DRUG), res(N_DIS), res(N_DRUG),
        ] + w_specs,
        out_specs=[
            dis_blk(NSTEP1), res(N_DRUG), dis_blk(NSTEP1), drug_blk(NSTEP1),
            dis_blk(NSTEP1), res(N_DRUG), dis_blk(NSTEP1), drug_blk(NSTEP1),
            pl.BlockSpec((db1, N_DRUG), lambda i: (i, 0)),
            pl.BlockSpec((db1, N_DIS), lambda i: (i, 0)),
            pl.BlockSpec((ub1, N_DRUG), lambda i: (i, 0)),
        ],
        out_shape=[
            shp(N_DIS, DIM), shp(N_DRUG, DIM), shp(N_DIS, DIM), shp(N_DRUG, DIM),
            shp(N_DIS, DIM, _BF16), shp(N_DRUG, DIM, _BF16),
            shp(N_DIS, DIM, _BF16), shp(N_DRUG, DIM, _BF16),
            shp(N_DIS, N_DRUG, _BF16), shp(N_DIS, N_DIS, _BF16),
            shp(N_DRUG, N_DRUG, _BF16),
        ],
    )(interact_mat, v_edge, u_edge, dis_emb, dr_emb, di_emb_sim, dr_emb_sim,
      dilw, drlw, lat)
    dis1, dr1, dsim1, usim1 = outs1[0:4]
    dis1b, dr1b, dsim1b, usim1b = outs1[4:8]
    a_b, v_b, u_b = outs1[8:11]

    db2, ub2 = N_DIS // NSTEP2, N_DRUG // NSTEP2
    w2_specs = [
        pl.BlockSpec((db2, NFAC), lambda i: (i, 0)),
        pl.BlockSpec((N_DRUG, NFAC), lambda i: (0, 0)),
        pl.BlockSpec((NFAC, DIM), lambda i: (0, 0)),
    ]
    outs2 = pl.pallas_call(
        _hop2_body,
        grid=(NSTEP2,),
        in_specs=[
            pl.BlockSpec((db2, N_DRUG), lambda i: (i, 0)),
            pl.BlockSpec((db2, N_DIS), lambda i: (i, 0)),
            pl.BlockSpec((ub2, N_DRUG), lambda i: (i, 0)),
            dis_blk(NSTEP2), res(N_DRUG), res(N_DIS), res(N_DRUG),
        ] + w2_specs,
        out_specs=[
            dis_blk(NSTEP2), res(N_DRUG), dis_blk(NSTEP2), drug_blk(NSTEP2),
            res(N_DRUG),
            res(N_DRUG), dis_blk(NSTEP2), drug_blk(NSTEP2),
        ],
        out_shape=[
            shp(N_DIS, DIM), shp(N_DRUG, DIM), shp(N_DIS, DIM), shp(N_DRUG, DIM),
            shp(N_DRUG, DIM),
            shp(N_DRUG, DIM, _BF16), shp(N_DIS, DIM, _BF16),
            shp(N_DRUG, DIM, _BF16),
        ],
    )(a_b, v_b, u_b, dis1b, dr1b, dsim1b, usim1b, dilw, drlw, lat)
    dis2, dr2, dsim2, usim2, dr3 = outs2[0:5]
    dr2b, dsim2b, usim2b = outs2[5:8]

    db3, ub3 = N_DIS // NSTEP3, N_DRUG // NSTEP3
    outs3 = pl.pallas_call(
        _hop3_body,
        grid=(NSTEP3,),
        in_specs=[
            pl.BlockSpec((db3, N_DRUG), lambda i: (i, 0)),
            pl.BlockSpec((db3, N_DIS), lambda i: (i, 0)),
            pl.BlockSpec((ub3, N_DRUG), lambda i: (i, 0)),
            res(N_DRUG), res(N_DIS), res(N_DRUG),
            pl.BlockSpec((db3, NFAC), lambda i: (i, 0)),
            pl.BlockSpec((NFAC, DIM), lambda i: (0, 0)),
        ] + [dis_blk(NSTEP3)] * 6 + [drug_blk(NSTEP3)] * 7,
        out_specs=[
            pl.BlockSpec((db3, 8 * DIM), lambda i: (i, 0)),
            pl.BlockSpec((ub3, 8 * DIM), lambda i: (i, 0)),
        ],
        out_shape=[shp(N_DIS, 8 * DIM), shp(N_DRUG, 8 * DIM)],
    )(a_b, v_b, u_b, dr2b, dsim2b, usim2b, dilw, lat,
      dis_emb, di_emb_sim, dis1, dsim1, dis2, dsim2,
      dr_emb, dr_emb_sim, dr1, usim1, dr2, usim2, dr3)
    dis_res, drug_res = outs3

    return (dis_res, drug_res, jnp.float32(0.0))
